# baseline (speedup 1.0000x reference)
import functools

import jax
import jax.numpy as jnp
from jax import lax
from jax.experimental import pallas as pl
from jax.experimental.pallas import tpu as pltpu
from jax.experimental.pallas import tpu_sc as plsc

_B = 16384
_K = 10
_HID = 64
_NN = 1_000_000
_NC = 2
_NS = 16
_NW = _NC * _NS
_RW = _B // _NW
_SEG = 32768
_TBL = _NW * _SEG
_SUB = 64
_IC = 128

_mesh = plsc.VectorSubcoreMesh(
    core_axis_name="c", subcore_axis_name="s", num_cores=_NC, num_subcores=_NS
)
_params = pltpu.CompilerParams(use_tc_tiling_on_sc=False,
                               needs_layout_passes=False)


def _wid():
    return lax.axis_index("s") * _NC + lax.axis_index("c")


def _k1_body(head_hbm, rel_hbm, node_hbm, simw_hbm, simn_hbm, dc_hbm,
             upd_hbm, tbl_hbm,
             headall, relall, dcb, nbrl, wvl, flat_idx, oldb, nbremb, updb,
             tslice, sem0, sem1, sem2):
    w = _wid()
    lane = lax.iota(jnp.int32, 16)
    b0 = pl.multiple_of(w * _RW, _RW)

    pltpu.sync_copy(head_hbm, headall)
    pltpu.sync_copy(rel_hbm, relall)

    cps = [
        pltpu.async_copy(
            dc_hbm.at[headall.at[pl.ds(pl.multiple_of(b0 + cc * _IC, _IC), _IC)]],
            dcb.at[pl.ds(cc * _IC, _IC)], sem0)
        for cc in range(_RW // _IC)
    ]
    for cp in cps:
        cp.wait()

    for s in range(_RW // _SUB):
        off = pl.multiple_of(b0 + s * _SUB, _SUB)
        hsl = headall.at[pl.ds(off, _SUB)]
        c3 = pltpu.async_copy(node_hbm.at[hsl], oldb, sem2)

        def build(j, carry):
            p = j * 16 + lane
            r = lax.shift_right_logical(p * 6554, 16)
            k = p - r * _K
            hv = plsc.load_gather(headall, [b0 + s * _SUB + r])
            plsc.store_scatter(flat_idx, [p], k * _NN + hv)
            return carry
        lax.fori_loop(0, (_SUB * _K) // 16, build, 0)

        gs = []
        for cc in range((_SUB * _K) // _IC):
            sl = pl.ds(cc * _IC, _IC)
            gs.append(pltpu.async_copy(simn_hbm.at[flat_idx.at[sl]],
                                       nbrl.at[sl], sem0))
            gs.append(pltpu.async_copy(simw_hbm.at[flat_idx.at[sl]],
                                       wvl.at[sl], sem1))
        for g in gs:
            g.wait()

        gs = [
            pltpu.async_copy(
                node_hbm.at[nbrl.at[pl.ds(cc * _IC, _IC)]],
                nbremb.at[pl.ds(cc * _IC, _IC), :], sem0)
            for cc in range((_SUB * _K) // _IC)
        ]
        for g in gs:
            g.wait()
        c3.wait()

        def row(r, carry):
            q = r * _K
            rsp = jnp.full((16,), r, jnp.int32)
            accs = [jnp.zeros((16,), jnp.float32) for _ in range(_HID // 16)]
            for k in range(_K):
                wk = plsc.load_gather(wvl, [jnp.full((16,), q + k, jnp.int32)])
                for h in range(_HID // 16):
                    accs[h] = accs[h] + wk * nbremb[q + k, pl.ds(h * 16, 16)]
            cv = plsc.load_gather(dcb, [jnp.full((16,), s * _SUB + r, jnp.int32)])
            rlv = plsc.load_gather(relall, [jnp.full((16,), b0 + s * _SUB + r,
                                                     jnp.int32)])
            mf = jnp.where((rlv >= 2) & (rlv <= 4), 1.0, 0.0)
            for h in range(_HID // 16):
                ov = oldb[r, pl.ds(h * 16, 16)]
                nv = cv * accs[h] + (1.0 - cv) * ov
                updb[r, pl.ds(h * 16, 16)] = mf * nv + (1.0 - mf) * ov
            return carry
        lax.fori_loop(0, _SUB, row, 0)
        pltpu.sync_copy(updb, upd_hbm.at[pl.ds(off, _SUB)])

    zv = jnp.zeros((16,), jnp.int32)

    def zero(j, carry):
        plsc.store_scatter(tslice, [j * 16 + lane], zv)
        return carry
    lax.fori_loop(0, _SEG // 16, zero, 0)

    def vbody(v, carry):
        i16 = v * 16 + lane
        h = plsc.load_gather(headall, [i16])
        rv = plsc.load_gather(relall, [i16])
        m = ((rv >= 2) & (rv <= 4)).astype(jnp.int32)
        code = 2 * i16 + m + 1
        _, lastm = plsc.scan_count(h)
        in_rng = lax.shift_right_logical(h, 15) == jnp.full((16,), w, jnp.int32)
        plsc.store_scatter(tslice, [h & (_SEG - 1)], code, mask=lastm & in_rng)
        return carry
    lax.fori_loop(0, _B // 16, vbody, 0)

    pltpu.sync_copy(
        tslice, tbl_hbm.at[pl.ds(pl.multiple_of(w * _SEG, _SEG), _SEG)])


def _k2_body(head_hbm, rel_hbm, tail_hbm, node_hbm, relemb_hbm, upd_hbm,
             tbl_hbm, out_hbm,
             hb, rb, tb, ghb, gtb, idxh, idxt, hrows, tnew, told, rtab, scb,
             sem0, sem1, sem2):
    w = _wid()
    lane = lax.iota(jnp.int32, 16)
    b0 = pl.multiple_of(w * _RW, _RW)

    pltpu.sync_copy(head_hbm.at[pl.ds(b0, _RW)], hb)
    pltpu.sync_copy(rel_hbm.at[pl.ds(b0, _RW)], rb)
    pltpu.sync_copy(tail_hbm.at[pl.ds(b0, _RW)], tb)
    pltpu.sync_copy(relemb_hbm, rtab)

    cps = []
    for cc in range(_RW // _IC):
        sl = pl.ds(cc * _IC, _IC)
        cps.append(pltpu.async_copy(tbl_hbm.at[hb.at[sl]], ghb.at[sl], sem0))
        cps.append(pltpu.async_copy(tbl_hbm.at[tb.at[sl]], gtb.at[sl], sem1))
    for cp in cps:
        cp.wait()

    def ibody(v, carry):
        i16 = v * 16 + lane
        gh = plsc.load_gather(ghb, [i16])
        wrh = jnp.maximum(lax.shift_right_arithmetic(gh - 1, 1), 0)
        plsc.store_scatter(idxh, [i16], wrh)
        gt = plsc.load_gather(gtb, [i16])
        wrt = jnp.maximum(lax.shift_right_arithmetic(gt - 1, 1), 0)
        plsc.store_scatter(idxt, [i16], wrt)
        return carry
    lax.fori_loop(0, _RW // 16, ibody, 0)

    cps = []
    for cc in range(_RW // _IC):
        sl = pl.ds(cc * _IC, _IC)
        cps.append(pltpu.async_copy(upd_hbm.at[idxh.at[sl]],
                                    hrows.at[sl, :], sem0))
        cps.append(pltpu.async_copy(upd_hbm.at[idxt.at[sl]],
                                    tnew.at[sl, :], sem1))
        cps.append(pltpu.async_copy(node_hbm.at[tb.at[sl]],
                                    told.at[sl, :], sem2))
    for cp in cps:
        cp.wait()

    lane16 = lax.iota(jnp.int32, 16)

    def row(r, carry):
        rsp = jnp.full((16,), r, jnp.int32)
        relv = plsc.load_gather(rb, [rsp])
        sf = jnp.where(plsc.load_gather(gtb, [rsp]) > 0, 1.0, 0.0)
        acc = jnp.zeros((16,), jnp.float32)
        for h in range(_HID // 16):
            hv = hrows[r, pl.ds(h * 16, 16)]
            tv = sf * tnew[r, pl.ds(h * 16, 16)] + (1.0 - sf) * told[r, pl.ds(h * 16, 16)]
            rvv = plsc.load_gather(rtab, [relv * _HID + h * 16 + lane16])
            acc = acc + hv * rvv * tv
        tot = jnp.full((16,), jnp.sum(acc))
        plsc.store_scatter(scb, [rsp], tot, mask=lane16 == 0)
        return carry
    lax.fori_loop(0, _RW, row, 0)

    pltpu.sync_copy(scb, out_hbm.at[pl.ds(b0, _RW)])


_k1 = functools.partial(
    pl.kernel,
    out_type=(
        jax.ShapeDtypeStruct((_B, _HID), jnp.float32),
        jax.ShapeDtypeStruct((_TBL,), jnp.int32),
    ),
    mesh=_mesh,
    compiler_params=_params,
    scratch_types=[
        pltpu.VMEM((_B,), jnp.int32),
        pltpu.VMEM((_B,), jnp.int32),
        pltpu.VMEM((_RW,), jnp.float32),
        pltpu.VMEM((_SUB * _K,), jnp.int32),
        pltpu.VMEM((_SUB * _K,), jnp.float32),
        pltpu.VMEM((_SUB * _K,), jnp.int32),
        pltpu.VMEM((_SUB, _HID), jnp.float32),
        pltpu.VMEM((_SUB * _K, _HID), jnp.float32),
        pltpu.VMEM((_SUB, _HID), jnp.float32),
        pltpu.VMEM((_SEG,), jnp.int32),
        pltpu.SemaphoreType.DMA,
        pltpu.SemaphoreType.DMA,
        pltpu.SemaphoreType.DMA,
    ],
)(_k1_body)


_k2 = functools.partial(
    pl.kernel,
    out_type=jax.ShapeDtypeStruct((_B,), jnp.float32),
    mesh=_mesh,
    compiler_params=_params,
    scratch_types=[
        pltpu.VMEM((_RW,), jnp.int32),
        pltpu.VMEM((_RW,), jnp.int32),
        pltpu.VMEM((_RW,), jnp.int32),
        pltpu.VMEM((_RW,), jnp.int32),
        pltpu.VMEM((_RW,), jnp.int32),
        pltpu.VMEM((_RW,), jnp.int32),
        pltpu.VMEM((_RW,), jnp.int32),
        pltpu.VMEM((_RW, _HID), jnp.float32),
        pltpu.VMEM((_RW, _HID), jnp.float32),
        pltpu.VMEM((_RW, _HID), jnp.float32),
        pltpu.VMEM((8 * _HID,), jnp.float32),
        pltpu.VMEM((_RW,), jnp.float32),
        pltpu.SemaphoreType.DMA,
        pltpu.SemaphoreType.DMA,
        pltpu.SemaphoreType.DMA,
    ],
)(_k2_body)


def kernel(head_index, rel_type, tail_index, node_emb, rel_emb, sim_weights,
           sim_neighbors, disease_constant):
    upd, tbl = _k1(head_index, rel_type, node_emb,
                   sim_weights.T.reshape(-1), sim_neighbors.T.reshape(-1),
                   disease_constant)
    return _k2(head_index, rel_type, tail_index, node_emb,
               rel_emb.reshape(-1), upd, tbl)


# speedup vs baseline: 1.2866x; 1.2866x over previous
import functools

import jax
import jax.numpy as jnp
from jax import lax
from jax.experimental import pallas as pl
from jax.experimental.pallas import tpu as pltpu
from jax.experimental.pallas import tpu_sc as plsc

_B = 16384
_K = 10
_HID = 64
_NN = 1_000_000
_NC = 2
_NS = 16
_NW = _NC * _NS
_RW = _B // _NW
_SEG = 32768
_TBL = _NW * _SEG
_SUB = 64
_IC = 128

_mesh = plsc.VectorSubcoreMesh(
    core_axis_name="c", subcore_axis_name="s", num_cores=_NC, num_subcores=_NS
)
_params = pltpu.CompilerParams(use_tc_tiling_on_sc=False,
                               needs_layout_passes=False)


def _wid():
    return lax.axis_index("s") * _NC + lax.axis_index("c")


def _k1_body(head_hbm, rel_hbm, node_hbm, simw_hbm, simn_hbm, dc_hbm,
             upd_hbm, tbl_hbm,
             headall, relall, dcb, nbrl, wvl, flat_idx, oldb, nbremb, updb,
             tslice, sem0, sem1, sem2):
    w = _wid()
    lane = lax.iota(jnp.int32, 16)
    b0 = pl.multiple_of(w * _RW, _RW)

    pltpu.sync_copy(head_hbm, headall)
    pltpu.sync_copy(rel_hbm, relall)

    cps = [
        pltpu.async_copy(
            dc_hbm.at[headall.at[pl.ds(pl.multiple_of(b0 + cc * _IC, _IC), _IC)]],
            dcb.at[pl.ds(cc * _IC, _IC)], sem0)
        for cc in range(_RW // _IC)
    ]
    for cp in cps:
        cp.wait()

    for s in range(_RW // _SUB):
        off = pl.multiple_of(b0 + s * _SUB, _SUB)
        hsl = headall.at[pl.ds(off, _SUB)]
        c3 = pltpu.async_copy(node_hbm.at[hsl], oldb, sem2)

        def build(j, carry):
            p = j * 16 + lane
            r = lax.shift_right_logical(p * 6554, 16)
            k = p - r * _K
            hv = plsc.load_gather(headall, [b0 + s * _SUB + r])
            plsc.store_scatter(flat_idx, [p], hv * _K + k)
            return carry
        lax.fori_loop(0, (_SUB * _K) // 16, build, 0)

        gs = []
        for cc in range((_SUB * _K) // _IC):
            sl = pl.ds(cc * _IC, _IC)
            gs.append(pltpu.async_copy(simn_hbm.at[flat_idx.at[sl]],
                                       nbrl.at[sl], sem0))
            gs.append(pltpu.async_copy(simw_hbm.at[flat_idx.at[sl]],
                                       wvl.at[sl], sem1))
        gs.append(c3)
        for g in gs:
            g.wait()

        gs = [
            pltpu.async_copy(
                node_hbm.at[nbrl.at[pl.ds(cc * _IC, _IC)]],
                nbremb.at[pl.ds(cc * _IC, _IC), :], sem0)
            for cc in range((_SUB * _K) // _IC)
        ]
        for g in gs:
            g.wait()

        def row(r, carry):
            q = r * _K
            rsp = jnp.full((16,), r, jnp.int32)
            rlv = plsc.load_gather(relall, [jnp.full((16,), b0 + s * _SUB + r,
                                                     jnp.int32)])
            rl0 = rlv[0]

            @pl.when((rl0 >= 2) & (rl0 <= 4))
            def _():
                accs = [jnp.zeros((16,), jnp.float32)
                        for _ in range(_HID // 16)]
                for k in range(_K):
                    wk = plsc.load_gather(
                        wvl, [jnp.full((16,), q + k, jnp.int32)])
                    for h in range(_HID // 16):
                        accs[h] = accs[h] + wk * nbremb[q + k, pl.ds(h * 16, 16)]
                cv = plsc.load_gather(
                    dcb, [jnp.full((16,), s * _SUB + r, jnp.int32)])
                for h in range(_HID // 16):
                    ov = oldb[r, pl.ds(h * 16, 16)]
                    updb[r, pl.ds(h * 16, 16)] = cv * accs[h] + (1.0 - cv) * ov

            @pl.when(jnp.logical_not((rl0 >= 2) & (rl0 <= 4)))
            def _():
                for h in range(_HID // 16):
                    updb[r, pl.ds(h * 16, 16)] = oldb[r, pl.ds(h * 16, 16)]
            return carry
        lax.fori_loop(0, _SUB, row, 0)
        pltpu.sync_copy(updb, upd_hbm.at[pl.ds(off, _SUB)])

    zv = jnp.zeros((16,), jnp.int32)

    def zero(j, carry):
        plsc.store_scatter(tslice, [j * 16 + lane], zv)
        return carry
    lax.fori_loop(0, _SEG // 16, zero, 0)

    def vbody(v, carry):
        i16 = v * 16 + lane
        h = plsc.load_gather(headall, [i16])
        rv = plsc.load_gather(relall, [i16])
        m = ((rv >= 2) & (rv <= 4)).astype(jnp.int32)
        code = 2 * i16 + m + 1
        _, lastm = plsc.scan_count(h)
        in_rng = lax.shift_right_logical(h, 15) == jnp.full((16,), w, jnp.int32)
        plsc.store_scatter(tslice, [h & (_SEG - 1)], code, mask=lastm & in_rng)
        return carry
    lax.fori_loop(0, _B // 16, vbody, 0)

    pltpu.sync_copy(
        tslice, tbl_hbm.at[pl.ds(pl.multiple_of(w * _SEG, _SEG), _SEG)])


def _k2_body(head_hbm, rel_hbm, tail_hbm, node_hbm, relemb_hbm, upd_hbm,
             tbl_hbm, out_hbm,
             hb, rb, tb, ghb, gtb, idxh, idxt, hrows, tnew, told, rtab, scb,
             sem0, sem1, sem2):
    w = _wid()
    lane = lax.iota(jnp.int32, 16)
    b0 = pl.multiple_of(w * _RW, _RW)

    pltpu.sync_copy(head_hbm.at[pl.ds(b0, _RW)], hb)
    pltpu.sync_copy(rel_hbm.at[pl.ds(b0, _RW)], rb)
    pltpu.sync_copy(tail_hbm.at[pl.ds(b0, _RW)], tb)
    pltpu.sync_copy(relemb_hbm, rtab)

    cps = []
    for cc in range(_RW // _IC):
        sl = pl.ds(cc * _IC, _IC)
        cps.append(pltpu.async_copy(tbl_hbm.at[hb.at[sl]], ghb.at[sl], sem0))
        cps.append(pltpu.async_copy(tbl_hbm.at[tb.at[sl]], gtb.at[sl], sem1))
    for cp in cps:
        cp.wait()

    def ibody(v, carry):
        i16 = v * 16 + lane
        gh = plsc.load_gather(ghb, [i16])
        wrh = jnp.maximum(lax.shift_right_arithmetic(gh - 1, 1), 0)
        plsc.store_scatter(idxh, [i16], wrh)
        gt = plsc.load_gather(gtb, [i16])
        wrt = jnp.maximum(lax.shift_right_arithmetic(gt - 1, 1), 0)
        plsc.store_scatter(idxt, [i16], wrt)
        return carry
    lax.fori_loop(0, _RW // 16, ibody, 0)

    cps = []
    for cc in range(_RW // _IC):
        sl = pl.ds(cc * _IC, _IC)
        cps.append(pltpu.async_copy(upd_hbm.at[idxh.at[sl]],
                                    hrows.at[sl, :], sem0))
        cps.append(pltpu.async_copy(upd_hbm.at[idxt.at[sl]],
                                    tnew.at[sl, :], sem1))
        cps.append(pltpu.async_copy(node_hbm.at[tb.at[sl]],
                                    told.at[sl, :], sem2))
    for cp in cps:
        cp.wait()

    lane16 = lax.iota(jnp.int32, 16)

    def row(r, carry):
        rsp = jnp.full((16,), r, jnp.int32)
        relv = plsc.load_gather(rb, [rsp])
        sf = jnp.where(plsc.load_gather(gtb, [rsp]) > 0, 1.0, 0.0)
        acc = jnp.zeros((16,), jnp.float32)
        for h in range(_HID // 16):
            hv = hrows[r, pl.ds(h * 16, 16)]
            tv = sf * tnew[r, pl.ds(h * 16, 16)] + (1.0 - sf) * told[r, pl.ds(h * 16, 16)]
            rvv = plsc.load_gather(rtab, [relv * _HID + h * 16 + lane16])
            acc = acc + hv * rvv * tv
        tot = jnp.full((16,), jnp.sum(acc))
        plsc.store_scatter(scb, [rsp], tot, mask=lane16 == 0)
        return carry
    lax.fori_loop(0, _RW, row, 0)

    pltpu.sync_copy(scb, out_hbm.at[pl.ds(b0, _RW)])


_k1 = functools.partial(
    pl.kernel,
    out_type=(
        jax.ShapeDtypeStruct((_B, _HID), jnp.float32),
        jax.ShapeDtypeStruct((_TBL,), jnp.int32),
    ),
    mesh=_mesh,
    compiler_params=_params,
    scratch_types=[
        pltpu.VMEM((_B,), jnp.int32),
        pltpu.VMEM((_B,), jnp.int32),
        pltpu.VMEM((_RW,), jnp.float32),
        pltpu.VMEM((_SUB * _K,), jnp.int32),
        pltpu.VMEM((_SUB * _K,), jnp.float32),
        pltpu.VMEM((_SUB * _K,), jnp.int32),
        pltpu.VMEM((_SUB, _HID), jnp.float32),
        pltpu.VMEM((_SUB * _K, _HID), jnp.float32),
        pltpu.VMEM((_SUB, _HID), jnp.float32),
        pltpu.VMEM((_SEG,), jnp.int32),
        pltpu.SemaphoreType.DMA,
        pltpu.SemaphoreType.DMA,
        pltpu.SemaphoreType.DMA,
    ],
)(_k1_body)


_k2 = functools.partial(
    pl.kernel,
    out_type=jax.ShapeDtypeStruct((_B,), jnp.float32),
    mesh=_mesh,
    compiler_params=_params,
    scratch_types=[
        pltpu.VMEM((_RW,), jnp.int32),
        pltpu.VMEM((_RW,), jnp.int32),
        pltpu.VMEM((_RW,), jnp.int32),
        pltpu.VMEM((_RW,), jnp.int32),
        pltpu.VMEM((_RW,), jnp.int32),
        pltpu.VMEM((_RW,), jnp.int32),
        pltpu.VMEM((_RW,), jnp.int32),
        pltpu.VMEM((_RW, _HID), jnp.float32),
        pltpu.VMEM((_RW, _HID), jnp.float32),
        pltpu.VMEM((_RW, _HID), jnp.float32),
        pltpu.VMEM((8 * _HID,), jnp.float32),
        pltpu.VMEM((_RW,), jnp.float32),
        pltpu.SemaphoreType.DMA,
        pltpu.SemaphoreType.DMA,
        pltpu.SemaphoreType.DMA,
    ],
)(_k2_body)


def kernel(head_index, rel_type, tail_index, node_emb, rel_emb, sim_weights,
           sim_neighbors, disease_constant):
    upd, tbl = _k1(head_index, rel_type, node_emb, sim_weights.reshape(-1),
                   sim_neighbors.reshape(-1), disease_constant)
    return _k2(head_index, rel_type, tail_index, node_emb,
               rel_emb.reshape(-1), upd, tbl)


# speedup vs baseline: 1.2890x; 1.0019x over previous
import functools

import jax
import jax.numpy as jnp
from jax import lax
from jax.experimental import pallas as pl
from jax.experimental.pallas import tpu as pltpu
from jax.experimental.pallas import tpu_sc as plsc

_B = 16384
_K = 10
_HID = 64
_NN = 1_000_000
_NC = 2
_NS = 16
_NW = _NC * _NS
_RW = _B // _NW
_SEG = 32768
_TBL = _NW * _SEG
_SUB = 64
_IC = 128

_mesh = plsc.VectorSubcoreMesh(
    core_axis_name="c", subcore_axis_name="s", num_cores=_NC, num_subcores=_NS
)
_params = pltpu.CompilerParams(use_tc_tiling_on_sc=False,
                               needs_layout_passes=False)


def _wid():
    return lax.axis_index("s") * _NC + lax.axis_index("c")


def _k1_body(head_hbm, rel_hbm, node_hbm, simw_hbm, simn_hbm, dc_hbm,
             upd_hbm, tbl_hbm,
             headall, relall, dcb, nbrl, wvl, flat_idx, oldb, nbremb, updb,
             tslice, sem0, sem1, sem2):
    w = _wid()
    lane = lax.iota(jnp.int32, 16)
    b0 = pl.multiple_of(w * _RW, _RW)

    pltpu.sync_copy(head_hbm, headall)
    pltpu.sync_copy(rel_hbm, relall)

    pltpu.async_copy(dc_hbm.at[headall.at[pl.ds(b0, _RW)]], dcb, sem0).wait()

    for s in range(_RW // _SUB):
        off = pl.multiple_of(b0 + s * _SUB, _SUB)
        hsl = headall.at[pl.ds(off, _SUB)]
        c3 = pltpu.async_copy(node_hbm.at[hsl], oldb, sem2)

        def build(j, carry):
            p = j * 16 + lane
            r = lax.shift_right_logical(p * 6554, 16)
            k = p - r * _K
            hv = plsc.load_gather(headall, [b0 + s * _SUB + r])
            plsc.store_scatter(flat_idx, [p], hv * _K + k)
            return carry
        lax.fori_loop(0, (_SUB * _K) // 16, build, 0)

        g1 = pltpu.async_copy(simn_hbm.at[flat_idx], nbrl, sem0)
        g2 = pltpu.async_copy(simw_hbm.at[flat_idx], wvl, sem1)
        g1.wait()
        g2.wait()
        c3.wait()
        pltpu.async_copy(node_hbm.at[nbrl], nbremb, sem0).wait()

        def row(r, carry):
            q = r * _K
            rsp = jnp.full((16,), r, jnp.int32)
            rlv = plsc.load_gather(relall, [jnp.full((16,), b0 + s * _SUB + r,
                                                     jnp.int32)])
            rl0 = rlv[0]

            @pl.when((rl0 >= 2) & (rl0 <= 4))
            def _():
                accs = [jnp.zeros((16,), jnp.float32)
                        for _ in range(_HID // 16)]
                for k in range(_K):
                    wk = plsc.load_gather(
                        wvl, [jnp.full((16,), q + k, jnp.int32)])
                    for h in range(_HID // 16):
                        accs[h] = accs[h] + wk * nbremb[q + k, pl.ds(h * 16, 16)]
                cv = plsc.load_gather(
                    dcb, [jnp.full((16,), s * _SUB + r, jnp.int32)])
                for h in range(_HID // 16):
                    ov = oldb[r, pl.ds(h * 16, 16)]
                    updb[r, pl.ds(h * 16, 16)] = cv * accs[h] + (1.0 - cv) * ov

            @pl.when(jnp.logical_not((rl0 >= 2) & (rl0 <= 4)))
            def _():
                for h in range(_HID // 16):
                    updb[r, pl.ds(h * 16, 16)] = oldb[r, pl.ds(h * 16, 16)]
            return carry
        lax.fori_loop(0, _SUB, row, 0)
        pltpu.sync_copy(updb, upd_hbm.at[pl.ds(off, _SUB)])

    zv = jnp.zeros((16,), jnp.int32)

    def zero(j, carry):
        plsc.store_scatter(tslice, [j * 16 + lane], zv)
        return carry
    lax.fori_loop(0, _SEG // 16, zero, 0)

    def vbody(v, carry):
        i16 = v * 16 + lane
        h = plsc.load_gather(headall, [i16])
        rv = plsc.load_gather(relall, [i16])
        m = ((rv >= 2) & (rv <= 4)).astype(jnp.int32)
        code = 2 * i16 + m + 1
        _, lastm = plsc.scan_count(h)
        in_rng = lax.shift_right_logical(h, 15) == jnp.full((16,), w, jnp.int32)
        plsc.store_scatter(tslice, [h & (_SEG - 1)], code, mask=lastm & in_rng)
        return carry
    lax.fori_loop(0, _B // 16, vbody, 0)

    pltpu.sync_copy(
        tslice, tbl_hbm.at[pl.ds(pl.multiple_of(w * _SEG, _SEG), _SEG)])


def _k2_body(head_hbm, rel_hbm, tail_hbm, node_hbm, relemb_hbm, upd_hbm,
             tbl_hbm, out_hbm,
             hb, rb, tb, ghb, gtb, idxh, idxt, hrows, tnew, told, rtab, scb,
             sem0, sem1, sem2):
    w = _wid()
    lane = lax.iota(jnp.int32, 16)
    b0 = pl.multiple_of(w * _RW, _RW)

    pltpu.sync_copy(head_hbm.at[pl.ds(b0, _RW)], hb)
    pltpu.sync_copy(rel_hbm.at[pl.ds(b0, _RW)], rb)
    pltpu.sync_copy(tail_hbm.at[pl.ds(b0, _RW)], tb)
    pltpu.sync_copy(relemb_hbm, rtab)

    c1 = pltpu.async_copy(tbl_hbm.at[hb], ghb, sem0)
    c2 = pltpu.async_copy(tbl_hbm.at[tb], gtb, sem1)
    c1.wait()
    c2.wait()

    def ibody(v, carry):
        i16 = v * 16 + lane
        gh = plsc.load_gather(ghb, [i16])
        wrh = jnp.maximum(lax.shift_right_arithmetic(gh - 1, 1), 0)
        plsc.store_scatter(idxh, [i16], wrh)
        gt = plsc.load_gather(gtb, [i16])
        wrt = jnp.maximum(lax.shift_right_arithmetic(gt - 1, 1), 0)
        plsc.store_scatter(idxt, [i16], wrt)
        return carry
    lax.fori_loop(0, _RW // 16, ibody, 0)

    c1 = pltpu.async_copy(upd_hbm.at[idxh], hrows, sem0)
    c2 = pltpu.async_copy(upd_hbm.at[idxt], tnew, sem1)
    c3 = pltpu.async_copy(node_hbm.at[tb], told, sem2)
    c1.wait()
    c2.wait()
    c3.wait()

    lane16 = lax.iota(jnp.int32, 16)

    def row(r, carry):
        rsp = jnp.full((16,), r, jnp.int32)
        relv = plsc.load_gather(rb, [rsp])
        sf = jnp.where(plsc.load_gather(gtb, [rsp]) > 0, 1.0, 0.0)
        acc = jnp.zeros((16,), jnp.float32)
        for h in range(_HID // 16):
            hv = hrows[r, pl.ds(h * 16, 16)]
            tv = sf * tnew[r, pl.ds(h * 16, 16)] + (1.0 - sf) * told[r, pl.ds(h * 16, 16)]
            rvv = plsc.load_gather(rtab, [relv * _HID + h * 16 + lane16])
            acc = acc + hv * rvv * tv
        tot = jnp.full((16,), jnp.sum(acc))
        plsc.store_scatter(scb, [rsp], tot, mask=lane16 == 0)
        return carry
    lax.fori_loop(0, _RW, row, 0)

    pltpu.sync_copy(scb, out_hbm.at[pl.ds(b0, _RW)])


_k1 = functools.partial(
    pl.kernel,
    out_type=(
        jax.ShapeDtypeStruct((_B, _HID), jnp.float32),
        jax.ShapeDtypeStruct((_TBL,), jnp.int32),
    ),
    mesh=_mesh,
    compiler_params=_params,
    scratch_types=[
        pltpu.VMEM((_B,), jnp.int32),
        pltpu.VMEM((_B,), jnp.int32),
        pltpu.VMEM((_RW,), jnp.float32),
        pltpu.VMEM((_SUB * _K,), jnp.int32),
        pltpu.VMEM((_SUB * _K,), jnp.float32),
        pltpu.VMEM((_SUB * _K,), jnp.int32),
        pltpu.VMEM((_SUB, _HID), jnp.float32),
        pltpu.VMEM((_SUB * _K, _HID), jnp.float32),
        pltpu.VMEM((_SUB, _HID), jnp.float32),
        pltpu.VMEM((_SEG,), jnp.int32),
        pltpu.SemaphoreType.DMA,
        pltpu.SemaphoreType.DMA,
        pltpu.SemaphoreType.DMA,
    ],
)(_k1_body)


_k2 = functools.partial(
    pl.kernel,
    out_type=jax.ShapeDtypeStruct((_B,), jnp.float32),
    mesh=_mesh,
    compiler_params=_params,
    scratch_types=[
        pltpu.VMEM((_RW,), jnp.int32),
        pltpu.VMEM((_RW,), jnp.int32),
        pltpu.VMEM((_RW,), jnp.int32),
        pltpu.VMEM((_RW,), jnp.int32),
        pltpu.VMEM((_RW,), jnp.int32),
        pltpu.VMEM((_RW,), jnp.int32),
        pltpu.VMEM((_RW,), jnp.int32),
        pltpu.VMEM((_RW, _HID), jnp.float32),
        pltpu.VMEM((_RW, _HID), jnp.float32),
        pltpu.VMEM((_RW, _HID), jnp.float32),
        pltpu.VMEM((8 * _HID,), jnp.float32),
        pltpu.VMEM((_RW,), jnp.float32),
        pltpu.SemaphoreType.DMA,
        pltpu.SemaphoreType.DMA,
        pltpu.SemaphoreType.DMA,
    ],
)(_k2_body)


def kernel(head_index, rel_type, tail_index, node_emb, rel_emb, sim_weights,
           sim_neighbors, disease_constant):
    upd, tbl = _k1(head_index, rel_type, node_emb, sim_weights.reshape(-1),
                   sim_neighbors.reshape(-1), disease_constant)
    return _k2(head_index, rel_type, tail_index, node_emb,
               rel_emb.reshape(-1), upd, tbl)
